# baseline (speedup 1.0000x reference)
import jax
import jax.numpy as jnp
from jax.experimental import pallas as pl
from jax.experimental.pallas import tpu as pltpu
from jax.experimental.pallas import tpu_sc as plsc

B, K, C, H, W = 8, 20000, 64, 256, 256
HW = H * W
NT = 1264
LCAP = 1280
GB = 128
CHUNK = 16384
NDUMP = 64
F2P23 = 8388608.0


def _sc_body(feat_hbm, xy_hbm, base_hbm, xy_v, idlist, rowlist, gbuf,
             zero_v, cmap):
    c = jax.lax.axis_index("c")
    s = jax.lax.axis_index("s")
    lanes = jax.lax.iota(jnp.int32, 16)
    zeros16 = jnp.zeros((16,), jnp.int32)
    ones16 = jnp.ones((16,), jnp.int32)

    def _z(r, carry):
        for c4 in range(4):
            zero_v[r, pl.ds(c4 * 16, 16)] = jnp.zeros((16,), jnp.float32)
        return carry
    jax.lax.fori_loop(0, 64, _z, 0)

    def batch_body(b, carry):
        off = jnp.minimum(s * NT, K - NT)
        inv = s * NT - off
        gbase = b * K + off

        pltpu.sync_copy(xy_hbm.at[pl.ds(gbase, NT), :], xy_v.at[pl.ds(0, NT)])

        def _pf(i, carry2):
            idlist[pl.ds(i * 16, 16)] = b * K + (((i * 16 + lanes) * 7) & 8191)
            rowlist[pl.ds(i * 16, 16)] = CHUNK + ((i + lanes) & (NDUMP - 1))
            return carry2
        jax.lax.fori_loop(0, 2 * LCAP // 16, _pf, 0)

        def _ix(v, cur):
            rows = lanes + v * 16
            x = plsc.load_gather(xy_v, [rows, zeros16])
            y = plsc.load_gather(xy_v, [rows, ones16])
            xr = (x + F2P23) - F2P23
            yr = (y + F2P23) - F2P23
            xi = jnp.clip(xr, 0.0, 255.0).astype(jnp.int32)
            yi = jnp.clip(yr, 0.0, 255.0).astype(jnp.int32)
            g = yi * 256 + xi
            valid = (rows >= inv) & (rows < NT)
            g = jnp.where(valid, g, -1)
            gid = gbase + rows
            new_cur = []
            for ci in range(2):
                cb = (c * 2 + ci) * CHUNK
                loc = g - cb
                m = (loc >= 0) & (loc < CHUNK)
                cnt = plsc.all_reduce_population_count(m)[0]
                dst = ci * LCAP + cur[ci]
                plsc.store_compressed(idlist.at[pl.ds(dst, 16)], gid, mask=m)
                plsc.store_compressed(rowlist.at[pl.ds(dst, 16)], loc, mask=m)
                new_cur.append(cur[ci] + cnt)
            return tuple(new_cur)
        counts = jax.lax.fori_loop(0, NT // 16 + 1, _ix,
                                   (jnp.int32(0), jnp.int32(0)))

        for ci in range(2):
            cb = (c * 2 + ci) * CHUNK
            nblk = (counts[ci] + (GB - 1)) // GB

            def _zz(z, carry2):
                pltpu.sync_copy(zero_v, cmap.at[pl.ds(s * 1024 + z * 64, 64)])
                return carry2
            jax.lax.fori_loop(0, 16, _zz, 0)
            plsc.subcore_barrier()

            def _blk(j, carry2):
                sofs = ci * LCAP + j * GB
                pltpu.sync_copy(feat_hbm.at[idlist.at[pl.ds(sofs, GB)]], gbuf)
                pltpu.sync_copy(gbuf, cmap.at[rowlist.at[pl.ds(sofs, GB)]],
                                add=True)
                return carry2
            jax.lax.fori_loop(0, nblk, _blk, 0)
            plsc.subcore_barrier()

            pltpu.sync_copy(cmap.at[pl.ds(s * 1024, 1024)],
                            base_hbm.at[b, pl.ds(cb + s * 1024, 1024), :])
            plsc.subcore_barrier()
        return carry
    jax.lax.fori_loop(0, B, batch_body, 0)


def _rasterize_sc(node_feat, node_xy):
    f = pl.kernel(
        _sc_body,
        out_type=jax.ShapeDtypeStruct((B, HW, C), jnp.float32),
        mesh=plsc.VectorSubcoreMesh(core_axis_name="c", subcore_axis_name="s"),
        compiler_params=pltpu.CompilerParams(needs_layout_passes=False,
                                             use_tc_tiling_on_sc=False),
        scratch_types=[
            pltpu.VMEM((LCAP, 2), jnp.float32),
            pltpu.VMEM((2 * LCAP,), jnp.int32),
            pltpu.VMEM((2 * LCAP,), jnp.int32),
            pltpu.VMEM((GB, C), jnp.float32),
            pltpu.VMEM((64, C), jnp.float32),
            pltpu.VMEM_SHARED((CHUNK + NDUMP, C), jnp.float32),
        ],
    )
    return f(node_feat.reshape(B * K, C), node_xy.reshape(B * K, 2))


def _mm_body(w_ref, x_ref, o_ref):
    w = w_ref[...]
    for r in range(8):
        blk = x_ref[0, pl.ds(r * 128, 128), :]
        a = jax.lax.dot_general(w, blk[:, 0:C],
                                dimension_numbers=(((1,), (1,)), ((), ())),
                                preferred_element_type=jnp.float32)
        b = jax.lax.dot_general(w, blk[:, C:2 * C],
                                dimension_numbers=(((1,), (1,)), ((), ())),
                                preferred_element_type=jnp.float32)
        o_ref[0, :, r, :] = jnp.stack([a, b], axis=-1).reshape(C, W)


def _conv1x1(base, W_obj):
    base128 = base.reshape(B, HW // 2, 2 * C)
    return pl.pallas_call(
        _mm_body,
        grid=(B, 32),
        in_specs=[
            pl.BlockSpec((C, C), lambda b, p: (0, 0)),
            pl.BlockSpec((1, 1024, 2 * C), lambda b, p: (b, p, 0)),
        ],
        out_specs=pl.BlockSpec((1, C, 8, W), lambda b, p: (b, 0, p, 0)),
        out_shape=jax.ShapeDtypeStruct((B, C, H, W), jnp.float32),
    )(W_obj, base128)


@jax.jit
def kernel(node_feat, node_xy, hw, W_obj):
    del hw
    base = _rasterize_sc(node_feat, node_xy)
    return _conv1x1(base, W_obj)


# speedup vs baseline: 32.0687x; 32.0687x over previous
import jax
import jax.numpy as jnp
from jax.experimental import pallas as pl
from jax.experimental.pallas import tpu as pltpu
from jax.experimental.pallas import tpu_sc as plsc

B, K, C, H, W = 8, 20000, 64, 256, 256
HW = H * W
NT = 1264
LCAP = 1280
GB = 128
CHUNK = 16384
NDUMP = 64
F2P23 = 8388608.0


def _sc_body(feat_hbm, xy_hbm, base_hbm, xy_v, idlist, rowlist, gbuf,
             zero_v, cmap):
    c = jax.lax.axis_index("c")
    s = jax.lax.axis_index("s")
    lanes = jax.lax.iota(jnp.int32, 16)
    zeros16 = jnp.zeros((16,), jnp.int32)
    ones16 = jnp.ones((16,), jnp.int32)

    def _z(r, carry):
        for c4 in range(4):
            zero_v[r, pl.ds(c4 * 16, 16)] = jnp.zeros((16,), jnp.float32)
        return carry
    jax.lax.fori_loop(0, 64, _z, 0)

    def batch_body(b, carry):
        off = jnp.minimum(s * NT, K - NT)
        inv = s * NT - off
        gbase = b * K + off

        pltpu.sync_copy(xy_hbm.at[pl.ds(gbase, NT), :], xy_v.at[pl.ds(0, NT)])

        def _pf(i, carry2):
            idlist[pl.ds(i * 16, 16)] = b * K + (((i * 16 + lanes) * 7) & 8191)
            rowlist[pl.ds(i * 16, 16)] = CHUNK + ((i + lanes) & (NDUMP - 1))
            return carry2
        jax.lax.fori_loop(0, 2 * LCAP // 16, _pf, 0)

        def _ix(v, cur):
            rows = lanes + v * 16
            x = plsc.load_gather(xy_v, [rows, zeros16])
            y = plsc.load_gather(xy_v, [rows, ones16])
            xr = (x + F2P23) - F2P23
            yr = (y + F2P23) - F2P23
            xi = jnp.clip(xr, 0.0, 255.0).astype(jnp.int32)
            yi = jnp.clip(yr, 0.0, 255.0).astype(jnp.int32)
            g = yi * 256 + xi
            valid = (rows >= inv) & (rows < NT)
            g = jnp.where(valid, g, -1)
            gid = gbase + rows
            new_cur = []
            for ci in range(2):
                cb = (c * 2 + ci) * CHUNK
                loc = g - cb
                m = (loc >= 0) & (loc < CHUNK)
                cnt = plsc.all_reduce_population_count(m)[0]
                dst = ci * LCAP + cur[ci]
                plsc.store_compressed(idlist.at[pl.ds(dst, 16)], gid, mask=m)
                plsc.store_compressed(rowlist.at[pl.ds(dst, 16)], loc, mask=m)
                new_cur.append(cur[ci] + cnt)
            return tuple(new_cur)
        counts = jax.lax.fori_loop(0, NT // 16 + 1, _ix,
                                   (jnp.int32(0), jnp.int32(0)))

        for ci in range(2):
            cb = (c * 2 + ci) * CHUNK
            nblk = (counts[ci] + (GB - 1)) // GB

            def _zz(z, carry2):
                pltpu.sync_copy(zero_v, cmap.at[pl.ds(s * 1024 + z * 64, 64)])
                return carry2
            jax.lax.fori_loop(0, 16, _zz, 0)
            plsc.subcore_barrier()

            def _blk(j, carry2):
                sofs = ci * LCAP + j * GB
                pltpu.sync_copy(feat_hbm.at[idlist.at[pl.ds(sofs, GB)]], gbuf)
                pltpu.sync_copy(gbuf, cmap.at[rowlist.at[pl.ds(sofs, GB)]],
                                add=True)
                return carry2
            jax.lax.fori_loop(0, nblk, _blk, 0)
            plsc.subcore_barrier()

            pltpu.sync_copy(cmap.at[pl.ds(s * 1024, 1024)],
                            base_hbm.at[b, pl.ds(cb + s * 1024, 1024), :])
            plsc.subcore_barrier()
        return carry
    jax.lax.fori_loop(0, B, batch_body, 0)


def _rasterize_sc(node_feat, node_xy):
    f = pl.kernel(
        _sc_body,
        out_type=jax.ShapeDtypeStruct((B, HW, C), jnp.float32),
        mesh=plsc.VectorSubcoreMesh(core_axis_name="c", subcore_axis_name="s"),
        compiler_params=pltpu.CompilerParams(needs_layout_passes=False,
                                             use_tc_tiling_on_sc=False),
        scratch_types=[
            pltpu.VMEM((LCAP, 2), jnp.float32),
            pltpu.VMEM((2 * LCAP,), jnp.int32),
            pltpu.VMEM((2 * LCAP,), jnp.int32),
            pltpu.VMEM((GB, C), jnp.float32),
            pltpu.VMEM((64, C), jnp.float32),
            pltpu.VMEM_SHARED((CHUNK + NDUMP, C), jnp.float32),
        ],
    )
    return f(node_feat.reshape(B * K, C), node_xy.reshape(B * K, 2))


def _mm_body(w_ref, x_ref, o_ref):
    w = w_ref[...]
    for r in range(8):
        blk = x_ref[0, pl.ds(r * W, W), :]
        o_ref[0, :, r, :] = jax.lax.dot_general(
            w, blk, dimension_numbers=(((1,), (1,)), ((), ())),
            preferred_element_type=jnp.float32)


def _conv1x1(base, W_obj):
    return pl.pallas_call(
        _mm_body,
        grid=(B, 32),
        in_specs=[
            pl.BlockSpec((C, C), lambda b, p: (0, 0)),
            pl.BlockSpec((1, 8 * W, C), lambda b, p: (b, p, 0)),
        ],
        out_specs=pl.BlockSpec((1, C, 8, W), lambda b, p: (b, 0, p, 0)),
        out_shape=jax.ShapeDtypeStruct((B, C, H, W), jnp.float32),
    )(W_obj, base)


@jax.jit
def kernel(node_feat, node_xy, hw, W_obj):
    del hw
    base = _rasterize_sc(node_feat, node_xy)
    return _conv1x1(base, W_obj)


# speedup vs baseline: 40.2095x; 1.2539x over previous
import jax
import jax.numpy as jnp
from jax.experimental import pallas as pl
from jax.experimental.pallas import tpu as pltpu
from jax.experimental.pallas import tpu_sc as plsc

B, K, C, H, W = 8, 20000, 64, 256, 256
HW = H * W
NT = 1264
LCAP = 1280
GB = 128
CHUNK = 16384
NDUMP = 64
F2P23 = 8388608.0


def _sc_body(feat_hbm, xy_hbm, base_hbm, xy_v, idlist, rowlist, gbuf,
             scat_idx, zero_v, cmap):
    c = jax.lax.axis_index("c")
    s = jax.lax.axis_index("s")
    lanes = jax.lax.iota(jnp.int32, 16)
    zeros16 = jnp.zeros((16,), jnp.int32)
    ones16 = jnp.ones((16,), jnp.int32)

    def _z(r, carry):
        for c4 in range(4):
            zero_v[r, pl.ds(c4 * 16, 16)] = jnp.zeros((16,), jnp.float32)
        return carry
    jax.lax.fori_loop(0, 64, _z, 0)

    def batch_body(b, carry):
        off = jnp.minimum(s * NT, K - NT)
        inv = s * NT - off
        gbase = b * K + off

        pltpu.sync_copy(xy_hbm.at[pl.ds(gbase, NT), :], xy_v.at[pl.ds(0, NT)])

        def _pf(i, carry2):
            idlist[pl.ds(i * 16, 16)] = b * K + (((i * 16 + lanes) * 7) & 8191)
            rowlist[pl.ds(i * 16, 16)] = CHUNK + ((i + lanes) & (NDUMP - 1))
            return carry2
        jax.lax.fori_loop(0, 2 * LCAP // 16, _pf, 0)

        def _ix(v, cur):
            rows = lanes + v * 16
            x = plsc.load_gather(xy_v, [rows, zeros16])
            y = plsc.load_gather(xy_v, [rows, ones16])
            xr = (x + F2P23) - F2P23
            yr = (y + F2P23) - F2P23
            xi = jnp.clip(xr, 0.0, 255.0).astype(jnp.int32)
            yi = jnp.clip(yr, 0.0, 255.0).astype(jnp.int32)
            g = yi * 256 + xi
            valid = (rows >= inv) & (rows < NT)
            g = jnp.where(valid, g, -1)
            gid = gbase + rows
            new_cur = []
            for ci in range(2):
                cb = (c * 2 + ci) * CHUNK
                loc = g - cb
                m = (loc >= 0) & (loc < CHUNK)
                cnt = plsc.all_reduce_population_count(m)[0]
                dst = ci * LCAP + cur[ci]
                plsc.store_compressed(idlist.at[pl.ds(dst, 16)], gid, mask=m)
                plsc.store_compressed(rowlist.at[pl.ds(dst, 16)], loc, mask=m)
                new_cur.append(cur[ci] + cnt)
            return tuple(new_cur)
        counts = jax.lax.fori_loop(0, NT // 16 + 1, _ix,
                                   (jnp.int32(0), jnp.int32(0)))

        for ci in range(2):
            cb = (c * 2 + ci) * CHUNK
            nblk = (counts[ci] + (GB - 1)) // GB

            def _zz(z, carry2):
                pltpu.sync_copy(zero_v, cmap.at[pl.ds(s * 1024 + z * 64, 64)])
                return carry2
            jax.lax.fori_loop(0, 16, _zz, 0)
            plsc.subcore_barrier()

            def _blk(j, carry2):
                sofs = ci * LCAP + j * GB
                pltpu.sync_copy(feat_hbm.at[idlist.at[pl.ds(sofs, GB)]], gbuf)
                def _cp(k, c3):
                    scat_idx[pl.ds(k * 16, 16)] = rowlist[pl.ds(sofs + k * 16,
                                                                16)]
                    return c3
                jax.lax.fori_loop(0, GB // 16, _cp, 0)
                pltpu.sync_copy(gbuf, cmap.at[scat_idx], add=True)
                return carry2
            jax.lax.fori_loop(0, nblk, _blk, 0)
            plsc.subcore_barrier()

            def _wo(rr4, carry3):
                rr = s * 4 + rr4
                dst = (cb // 2) + rr * 128
                pltpu.sync_copy(cmap.at[pl.ds(rr * 256, 128)],
                                base_hbm.at[b, pl.ds(dst, 128), pl.ds(0, C)])
                pltpu.sync_copy(cmap.at[pl.ds(rr * 256 + 128, 128)],
                                base_hbm.at[b, pl.ds(dst, 128), pl.ds(C, C)])
                return carry3
            jax.lax.fori_loop(0, 4, _wo, 0)
            plsc.subcore_barrier()
        return carry
    jax.lax.fori_loop(0, B, batch_body, 0)


def _rasterize_sc(node_feat, node_xy):
    f = pl.kernel(
        _sc_body,
        out_type=jax.ShapeDtypeStruct((B, HW // 2, 2 * C), jnp.float32),
        mesh=plsc.VectorSubcoreMesh(core_axis_name="c", subcore_axis_name="s"),
        compiler_params=pltpu.CompilerParams(needs_layout_passes=False,
                                             use_tc_tiling_on_sc=False),
        scratch_types=[
            pltpu.VMEM((LCAP, 2), jnp.float32),
            pltpu.VMEM((2 * LCAP,), jnp.int32),
            pltpu.VMEM((2 * LCAP,), jnp.int32),
            pltpu.VMEM((GB, C), jnp.float32),
            pltpu.VMEM((GB,), jnp.int32),
            pltpu.VMEM((64, C), jnp.float32),
            pltpu.VMEM_SHARED((CHUNK + NDUMP, C), jnp.float32),
        ],
    )
    return f(node_feat.reshape(B * K, C), node_xy.reshape(B * K, 2))


def _mm_body(w_ref, x_ref, o_ref):
    dn = (((1,), (1,)), ((), ()))
    for r in range(8):
        xb = x_ref[0, pl.ds(r * 128, 128), :]
        a = jax.lax.dot_general(w_ref[0], xb, dn,
                                preferred_element_type=jnp.float32)
        b = jax.lax.dot_general(w_ref[1], xb, dn,
                                preferred_element_type=jnp.float32)
        o_ref[0, :, r, :] = jnp.concatenate([a, b], axis=1)


def _conv1x1(base, W_obj):
    z = jnp.zeros((C, C), jnp.float32)
    wlh = jnp.stack([jnp.concatenate([W_obj, z], axis=1),
                     jnp.concatenate([z, W_obj], axis=1)])
    return pl.pallas_call(
        _mm_body,
        grid=(B, 32),
        in_specs=[
            pl.BlockSpec((2, C, 2 * C), lambda b, p: (0, 0, 0)),
            pl.BlockSpec((1, 1024, 2 * C), lambda b, p: (b, p, 0)),
        ],
        out_specs=pl.BlockSpec((1, C, 8, W), lambda b, p: (b, 0, p, 0)),
        out_shape=jax.ShapeDtypeStruct((B, C, H, W), jnp.float32),
    )(wlh, base)


@jax.jit
def kernel(node_feat, node_xy, hw, W_obj):
    del hw
    base = _rasterize_sc(node_feat, node_xy)
    return _conv1x1(base, W_obj)
